# Initial kernel scaffold; baseline (speedup 1.0000x reference)
import jax
import jax.numpy as jnp
from jax.experimental import pallas as pl


def kernel(obs, Wc1, bc1, Wcin, bcin, Wcout, bcout, Wc2, bc2, Wch, bch, Wl, bl, Wiq, biq, Wio, bio, Wxq, bxq, Wxo, bxo, Was, bas, W1, b1, W2, b2, Wah, bah, Wvh, bvh):
    raise NotImplementedError("write your pallas kernel here")


# speedup vs baseline: 1.8187x; 1.8187x over previous
import jax
import jax.numpy as jnp
from jax.experimental import pallas as pl
from jax.experimental.pallas import tpu as pltpu

N = 512
O = 1024
D = 1024
A = 64
NH = 16
DH = D // NH
F32 = jnp.float32
NEG = -1e30


def _dot(a, b, ta=False, tb=False):
    dn = (((0,) if ta else (1,), (1,) if tb else (0,)), ((), ()))
    return jax.lax.dot_general(a, b, dn, preferred_element_type=F32)


def _attn(qkv, mask, scale):
    q = qkv[:, :D]
    k = qkv[:, D:2 * D]
    v = qkv[:, 2 * D:]
    outs = []
    for h in range(NH):
        sl = slice(h * DH, (h + 1) * DH)
        lg = _dot(q[:, sl], k[:, sl], tb=True) * scale
        if mask is not None:
            lg = jnp.where(mask, lg, NEG)
        mx = jnp.max(lg, axis=1, keepdims=True)
        e = jnp.exp(lg - mx)
        if mask is not None:
            e = jnp.where(mask, e, 0.0)
        p = e / jnp.maximum(jnp.sum(e, axis=1, keepdims=True), 1e-30)
        outs.append(_dot(p, v[:, sl]))
    return jnp.concatenate(outs, axis=1)


def _k1a(obs, Wc1, bc1, Wcin, bcin, x_out, qkv_out):
    x = jnp.tanh(_dot(obs[...], Wc1[...], tb=True) + bc1[...])
    x_out[...] = x
    qkv_out[...] = _dot(x, Wcin[...], tb=True) + bcin[...]


def _k1b(x, qkv, Wcout, bcout, Wc2, bc2, Wch, bch, cm_out):
    o = _attn(qkv[...], None, 1.0 / (DH ** 0.5))
    h = _dot(o, Wcout[...], tb=True) + bcout[...]
    Wc2v = Wc2[...]
    z = jnp.tanh(_dot(x[...], Wc2v[:, :D], tb=True)
                 + _dot(h, Wc2v[:, D:], tb=True) + bc2[...])
    lg = _dot(z, Wch[...], tb=True) + bch[...]
    mx = jnp.max(lg, axis=1, keepdims=True)
    e = jnp.exp(lg - mx)
    cm_out[...] = e / jnp.sum(e, axis=1, keepdims=True)


def _k2a(obs, cm, Wl, bl, Wiq, biq, Wio, bio, Was, bas,
         local_out, intra_out, glob_out, onehot_out):
    local = jnp.tanh(_dot(obs[...], Wl[...], tb=True) + bl[...])
    local_out[...] = local
    cmv = cm[...]
    col = jax.lax.broadcasted_iota(jnp.int32, (N, N), 1)
    mx = jnp.max(cmv, axis=1, keepdims=True)
    assign = jnp.min(jnp.where(cmv >= mx, col, N), axis=1, keepdims=True)
    onehot = (col == assign).astype(F32)
    onehot_out[...] = onehot
    same = _dot(onehot, onehot, tb=True) > 0.5
    qkv = _dot(local, Wiq[...], tb=True) + biq[...]
    intra_mha = _dot(_attn(qkv, same, 1.0 / (DH ** 0.5)), Wio[...], tb=True) + bio[...]
    counts_mat = _dot(onehot, jnp.ones((N, D), F32), ta=True)
    counts_ag = _dot(onehot, counts_mat)
    intra_out[...] = jnp.where(counts_ag > 1.5, intra_mha, local)
    sc_full = _dot(local * Was[...], jnp.ones((D, N), F32))
    S = jnp.where(onehot > 0.5, sc_full, NEG)
    mxc = jnp.max(S, axis=0, keepdims=True)
    e = jnp.where(onehot > 0.5, jnp.exp(S - mxc), 0.0)
    w = e / jnp.maximum(jnp.sum(e, axis=0, keepdims=True), 1e-30)
    glob_out[...] = _dot(w, local, ta=True)


def _k2b(glob, onehot, Wxq, bxq, Wxo, bxo, inter_out):
    oh = onehot[...]
    counts_row = jnp.sum(oh, axis=0, keepdims=True)
    counts_full = _dot(oh, jnp.ones((N, N), F32), ta=True)
    mask = (counts_full > 0.5) & (counts_row > 0.5)
    qkv = _dot(glob[...], Wxq[...], tb=True) + bxq[...]
    io = _dot(_attn(qkv, mask, 1.0 / (DH ** 0.5)), Wxo[...], tb=True) + bxo[...]
    inter_out[...] = _dot(oh, io)


def _k3(local, inter, intra, W1, b1, W2, b2, Wah, bah, Wvh, bvh, a_out, v_out):
    W1v = W1[...]
    hh = jnp.tanh(_dot(local[...], W1v[:, :D], tb=True)
                  + _dot(inter[...], W1v[:, D:2 * D], tb=True)
                  + _dot(intra[...], W1v[:, 2 * D:], tb=True) + b1[...])
    hh = jnp.tanh(_dot(hh, W2[...], tb=True) + b2[...])
    la = _dot(hh, Wah[...], tb=True) + bah[...]
    mx = jnp.max(la, axis=1, keepdims=True)
    a_out[...] = la - (mx + jnp.log(jnp.sum(jnp.exp(la - mx), axis=1, keepdims=True)))
    v_out[...] = jnp.sum(hh * Wvh[...], axis=1, keepdims=True) + bvh[0, 0]


def _sds(*shapes):
    return tuple(jax.ShapeDtypeStruct(s, F32) for s in shapes)


def kernel(obs, Wc1, bc1, Wcin, bcin, Wcout, bcout, Wc2, bc2, Wch, bch,
           Wl, bl, Wiq, biq, Wio, bio, Wxq, bxq, Wxo, bxo, Was, bas,
           W1, b1, W2, b2, Wah, bah, Wvh, bvh):
    r = lambda b: b.reshape(1, -1)
    x, qkv = pl.pallas_call(
        _k1a, out_shape=_sds((N, D), (N, 3 * D)))(obs, Wc1, r(bc1), Wcin, r(bcin))
    cm = pl.pallas_call(
        _k1b, out_shape=_sds((N, N))[0])(
        x, qkv, Wcout, r(bcout), Wc2, r(bc2), Wch, r(bch))
    local, intra, glob, onehot = pl.pallas_call(
        _k2a, out_shape=_sds((N, D), (N, D), (N, D), (N, N)))(
        obs, cm, Wl, r(bl), Wiq, r(biq), Wio, r(bio), Was, r(bas))
    inter = pl.pallas_call(
        _k2b, out_shape=_sds((N, D))[0])(glob, onehot, Wxq, r(bxq), Wxo, r(bxo))
    a, v = pl.pallas_call(
        _k3, out_shape=_sds((N, A), (N, 1)))(
        local, inter, intra, W1, r(b1), W2, r(b2), Wah, r(bah), Wvh, r(bvh))
    return (cm, a, v)


# speedup vs baseline: 2.2112x; 1.2158x over previous
import jax
import jax.numpy as jnp
from jax.experimental import pallas as pl
from jax.experimental.pallas import tpu as pltpu

N = 512
O = 1024
D = 1024
A = 64
NH = 16
DH = D // NH
F32 = jnp.float32
NEG = -1e30


def _dot(a, b, ta=False, tb=False):
    dn = (((0,) if ta else (1,), (1,) if tb else (0,)), ((), ()))
    return jax.lax.dot_general(a, b, dn, preferred_element_type=F32)


def _attn(qkv, amask, scale):
    q = qkv[:, :D] * scale
    k = qkv[:, D:2 * D]
    v = qkv[:, 2 * D:]
    outs = []
    for h in range(NH):
        sl = slice(h * DH, (h + 1) * DH)
        lg = _dot(q[:, sl], k[:, sl], tb=True)
        if amask is not None:
            lg = lg + amask
        mx = jnp.max(lg, axis=1, keepdims=True)
        e = jnp.exp(lg - mx)
        p = e / jnp.maximum(jnp.sum(e, axis=1, keepdims=True), 1e-30)
        outs.append(_dot(p, v[:, sl]))
    return jnp.concatenate(outs, axis=1)


def _k1(obs, Wc1, bc1, Wcin, bcin, Wcout, bcout, Wc2, bc2, Wch, bch, cm_out):
    x = jnp.tanh(_dot(obs[...], Wc1[...], tb=True) + bc1[...])
    qkv = _dot(x, Wcin[...], tb=True) + bcin[...]
    o = _attn(qkv, None, 1.0 / (DH ** 0.5))
    h = _dot(o, Wcout[...], tb=True) + bcout[...]
    Wc2v = Wc2[...]
    z = jnp.tanh(_dot(x, Wc2v[:, :D], tb=True)
                 + _dot(h, Wc2v[:, D:], tb=True) + bc2[...])
    lg = _dot(z, Wch[...], tb=True) + bch[...]
    mx = jnp.max(lg, axis=1, keepdims=True)
    e = jnp.exp(lg - mx)
    cm_out[...] = e / jnp.sum(e, axis=1, keepdims=True)


def _k2(obs, cm, Wl, bl, Wiq, biq, Wio, bio, Wxq, bxq, Wxo, bxo, Was,
        local_out, intra_out, inter_out):
    local = jnp.tanh(_dot(obs[...], Wl[...], tb=True) + bl[...])
    local_out[...] = local
    cmv = cm[...]
    col = jax.lax.broadcasted_iota(jnp.int32, (N, N), 1)
    mx = jnp.max(cmv, axis=1, keepdims=True)
    assign = jnp.min(jnp.where(cmv >= mx, col, N), axis=1, keepdims=True)
    onehot = (col == assign).astype(F32)
    amask_same = jnp.where(_dot(onehot, onehot, tb=True) > 0.5, 0.0, NEG)
    qkv = _dot(local, Wiq[...], tb=True) + biq[...]
    intra_mha = _dot(_attn(qkv, amask_same, 1.0 / (DH ** 0.5)),
                     Wio[...], tb=True) + bio[...]
    counts_mat = _dot(onehot, jnp.ones((N, D), F32), ta=True)
    counts_ag = _dot(onehot, counts_mat)
    intra_out[...] = jnp.where(counts_ag > 1.5, intra_mha, local)
    sc_full = _dot(local * Was[...], jnp.ones((D, N), F32))
    S = jnp.where(onehot > 0.5, sc_full, NEG)
    mxc = jnp.max(S, axis=0, keepdims=True)
    e = jnp.where(onehot > 0.5, jnp.exp(S - mxc), 0.0)
    w = e / jnp.maximum(jnp.sum(e, axis=0, keepdims=True), 1e-30)
    glob = _dot(w, local, ta=True)
    counts_row = jnp.sum(onehot, axis=0, keepdims=True)
    counts_full = _dot(onehot, jnp.ones((N, N), F32), ta=True)
    amask_x = jnp.where((counts_full > 0.5) & (counts_row > 0.5), 0.0, NEG)
    qkv2 = _dot(glob, Wxq[...], tb=True) + bxq[...]
    io = _dot(_attn(qkv2, amask_x, 1.0 / (DH ** 0.5)),
              Wxo[...], tb=True) + bxo[...]
    inter_out[...] = _dot(onehot, io)


def _k3(local, inter, intra, W1, b1, W2, b2, Wah, bah, Wvh, bvh, a_out, v_out):
    W1v = W1[...]
    hh = jnp.tanh(_dot(local[...], W1v[:, :D], tb=True)
                  + _dot(inter[...], W1v[:, D:2 * D], tb=True)
                  + _dot(intra[...], W1v[:, 2 * D:], tb=True) + b1[...])
    hh = jnp.tanh(_dot(hh, W2[...], tb=True) + b2[...])
    la = _dot(hh, Wah[...], tb=True) + bah[...]
    mx = jnp.max(la, axis=1, keepdims=True)
    a_out[...] = la - (mx + jnp.log(jnp.sum(jnp.exp(la - mx), axis=1, keepdims=True)))
    v_out[...] = jnp.sum(hh * Wvh[...], axis=1, keepdims=True) + bvh[0, 0]


def _sds(*shapes):
    return tuple(jax.ShapeDtypeStruct(s, F32) for s in shapes)


def kernel(obs, Wc1, bc1, Wcin, bcin, Wcout, bcout, Wc2, bc2, Wch, bch,
           Wl, bl, Wiq, biq, Wio, bio, Wxq, bxq, Wxo, bxo, Was, bas,
           W1, b1, W2, b2, Wah, bah, Wvh, bvh):
    r = lambda b: b.reshape(1, -1)
    cm = pl.pallas_call(
        _k1, out_shape=_sds((N, N))[0])(
        obs, Wc1, r(bc1), Wcin, r(bcin), Wcout, r(bcout), Wc2, r(bc2),
        Wch, r(bch))
    local, intra, inter = pl.pallas_call(
        _k2, out_shape=_sds((N, D), (N, D), (N, D)))(
        obs, cm, Wl, r(bl), Wiq, r(biq), Wio, r(bio), Wxq, r(bxq),
        Wxo, r(bxo), Was)
    a, v = pl.pallas_call(
        _k3, out_shape=_sds((N, A), (N, 1)))(
        local, inter, intra, W1, r(b1), W2, r(b2), Wah, r(bah), Wvh, r(bvh))
    return (cm, a, v)


# speedup vs baseline: 2.7296x; 1.2345x over previous
import jax
import jax.numpy as jnp
from jax.experimental import pallas as pl
from jax.experimental.pallas import tpu as pltpu

N = 512
O = 1024
D = 1024
A = 64
NH = 16
DH = D // NH
F32 = jnp.float32
NEG = -1e30


def _dot(a, b, ta=False, tb=False):
    dn = (((0,) if ta else (1,), (1,) if tb else (0,)), ((), ()))
    return jax.lax.dot_general(a, b, dn, preferred_element_type=F32)


def _attn(qkv, amask, scale):
    q = qkv[:, :D] * scale
    k = qkv[:, D:2 * D]
    v = qkv[:, 2 * D:]
    outs = []
    for h in range(NH):
        sl = slice(h * DH, (h + 1) * DH)
        lg = _dot(q[:, sl], k[:, sl], tb=True)
        if amask is not None:
            lg = lg + amask
        mx = jnp.max(lg, axis=1, keepdims=True)
        e = jnp.exp(lg - mx)
        p = e / jnp.maximum(jnp.sum(e, axis=1, keepdims=True), 1e-30)
        outs.append(_dot(p, v[:, sl]))
    return jnp.concatenate(outs, axis=1)


def _mega(obs, bc1, bcin, bcout, bc2, bch, bl, biq, bio, bxq, bxo, Was,
          b1, b2, Wah, bah, Wvh, bvh, Wch,
          Wc1h, Wcinh, Wcouth, Wc2h, Wlh, Wiqh, Wioh, Wxqh, Wxoh,
          W1h, W2h,
          cm_out, a_out, v_out,
          bufA, bufB, bufc, bufd, bufe,
          sem):
    scale = 1.0 / (DH ** 0.5)
    cp_wc1 = pltpu.make_async_copy(Wc1h, bufc, sem.at[0])
    cp_wc1.start()
    cp_wcin = pltpu.make_async_copy(Wcinh, bufA, sem.at[1])
    cp_wcin.start()
    cp_wcout = pltpu.make_async_copy(Wcouth, bufd, sem.at[2])
    cp_wcout.start()
    cp_wc2 = pltpu.make_async_copy(Wc2h, bufe, sem.at[3])
    cp_wc2.start()
    cp_wiq = pltpu.make_async_copy(Wiqh, bufB, sem.at[5])
    cp_wiq.start()

    cp_wc1.wait()
    x = jnp.tanh(_dot(obs[...], bufc[...], tb=True) + bc1[...])
    cp_wio = pltpu.make_async_copy(Wioh, bufc, sem.at[6])
    cp_wio.start()
    cp_wcin.wait()
    qkv = _dot(x, bufA[...], tb=True) + bcin[...]
    cp_wxq = pltpu.make_async_copy(Wxqh, bufA, sem.at[7])
    cp_wxq.start()
    o = _attn(qkv, None, scale)
    cp_wcout.wait()
    h1 = _dot(o, bufd[...], tb=True) + bcout[...]
    cp_wl = pltpu.make_async_copy(Wlh, bufd, sem.at[4])
    cp_wl.start()
    cp_wc2.wait()
    wc2v = bufe[...]
    z = jnp.tanh(_dot(x, wc2v[:, :D], tb=True)
                 + _dot(h1, wc2v[:, D:], tb=True) + bc2[...])
    cp_w1c = pltpu.make_async_copy(W1h.at[:, 2 * D:], bufe.at[:, D:], sem.at[9])
    cp_w1c.start()
    cp_w2 = pltpu.make_async_copy(W2h, bufe.at[:, :D], sem.at[10])
    cp_w2.start()
    lg = _dot(z, Wch[...], tb=True) + bch[...]
    mx = jnp.max(lg, axis=1, keepdims=True)
    e = jnp.exp(lg - mx)
    cm = e / jnp.sum(e, axis=1, keepdims=True)
    cm_out[...] = cm

    cp_wl.wait()
    local = jnp.tanh(_dot(obs[...], bufd[...], tb=True) + bl[...])
    cp_wxo = pltpu.make_async_copy(Wxoh, bufd, sem.at[8])
    cp_wxo.start()
    col = jax.lax.broadcasted_iota(jnp.int32, (N, N), 1)
    mxr = jnp.max(cm, axis=1, keepdims=True)
    assign = jnp.min(jnp.where(cm >= mxr, col, N), axis=1, keepdims=True)
    onehot = (col == assign).astype(F32)
    amask_same = jnp.where(_dot(onehot, onehot, tb=True) > 0.5, 0.0, NEG)
    cp_wiq.wait()
    qkv2 = _dot(local, bufB[...], tb=True) + biq[...]
    cp_w1a = pltpu.make_async_copy(W1h.at[:, :D], bufB.at[:D, :], sem.at[11])
    cp_w1a.start()
    cp_w1b = pltpu.make_async_copy(W1h.at[:, D:2 * D], bufB.at[D:2 * D, :],
                                   sem.at[12])
    cp_w1b.start()
    cp_wio.wait()
    intra_mha = _dot(_attn(qkv2, amask_same, scale), bufc[...], tb=True) + bio[...]
    counts_row = jnp.sum(onehot, axis=0, keepdims=True)
    counts_ag = jnp.sum(onehot * counts_row, axis=1, keepdims=True)
    intra = jnp.where(counts_ag > 1.5, intra_mha, local)
    sc_full = _dot(local * Was[...], jnp.ones((D, N), F32))
    S = jnp.where(onehot > 0.5, sc_full, NEG)
    mxc = jnp.max(S, axis=0, keepdims=True)
    es = jnp.where(onehot > 0.5, jnp.exp(S - mxc), 0.0)
    w = es / jnp.maximum(jnp.sum(es, axis=0, keepdims=True), 1e-30)
    glob = _dot(w, local, ta=True)
    amask_x = jnp.where(counts_row > 0.5, 0.0, NEG)
    cp_wxq.wait()
    qkv3 = _dot(glob, bufA[...], tb=True) + bxq[...]
    cp_wxo.wait()
    io = _dot(_attn(qkv3, amask_x, scale), bufd[...], tb=True) + bxo[...]
    inter = _dot(onehot, io)

    cp_w1a.wait()
    cp_w1b.wait()
    cp_w1c.wait()
    hh = jnp.tanh(_dot(local, bufB[:D, :], tb=True)
                  + _dot(inter, bufB[D:2 * D, :], tb=True)
                  + _dot(intra, bufe[:, D:], tb=True) + b1[...])
    cp_w2.wait()
    hh = jnp.tanh(_dot(hh, bufe[:, :D], tb=True) + b2[...])
    la = _dot(hh, Wah[...], tb=True) + bah[...]
    mxa = jnp.max(la, axis=1, keepdims=True)
    a_out[...] = la - (mxa + jnp.log(jnp.sum(jnp.exp(la - mxa), axis=1,
                                             keepdims=True)))
    v_out[...] = jnp.sum(hh * Wvh[...], axis=1, keepdims=True) + bvh[0, 0]


def kernel(obs, Wc1, bc1, Wcin, bcin, Wcout, bcout, Wc2, bc2, Wch, bch,
           Wl, bl, Wiq, biq, Wio, bio, Wxq, bxq, Wxo, bxo, Was, bas,
           W1, b1, W2, b2, Wah, bah, Wvh, bvh):
    r = lambda b: b.reshape(1, -1)
    n_vmem_in = 19
    n_hbm_in = 11
    vspec = pl.BlockSpec(memory_space=pltpu.MemorySpace.VMEM)
    aspec = pl.BlockSpec(memory_space=pl.ANY)
    cm, a, v = pl.pallas_call(
        _mega,
        in_specs=[vspec] * n_vmem_in + [aspec] * n_hbm_in,
        out_shape=(jax.ShapeDtypeStruct((N, N), F32),
                   jax.ShapeDtypeStruct((N, A), F32),
                   jax.ShapeDtypeStruct((N, 1), F32)),
        scratch_shapes=[
            pltpu.VMEM((3 * D, D), F32),
            pltpu.VMEM((3 * D, D), F32),
            pltpu.VMEM((D, O), F32),
            pltpu.VMEM((D, D), F32),
            pltpu.VMEM((D, 2 * D), F32),
            pltpu.SemaphoreType.DMA((13,)),
        ],
    )(obs, r(bc1), r(bcin), r(bcout), r(bc2), r(bch), r(bl), r(biq),
      r(bio), r(bxq), r(bxo), Was, r(b1), r(b2), Wah, r(bah), Wvh, r(bvh),
      Wch,
      Wc1, Wcin, Wcout, Wc2, Wl, Wiq, Wio, Wxq, Wxo, W1, W2)
    return (cm, a, v)
